# baseline (speedup 1.0000x reference)
import jax
import jax.numpy as jnp
from jax import lax
from jax.experimental import pallas as pl
from jax.experimental.pallas import tpu as pltpu
from jax.experimental.pallas import tpu_sc as plsc

N = 10000
E = 160000
D = 256
DOUT = 128
HB = 8
C = 256

NPAD = 10240
HALF = 5120
NTS = 16
SH_ROWS = 5136
DUMP = 5120
CSH = 384
ECH = 64
ER = 2560
EPAD = ER * ECH
ERT = 160
NPT = 320
SLR = 16
NSLA = 10
CHK = 48
CMAX = 10288
PKB = 14
ESL = 32
NSL = 80

_MESH = plsc.VectorSubcoreMesh(core_axis_name="c", subcore_axis_name="s")


def _sca_body(h_hbm, src_hbm, dst_hbm, agg_hbm, deg_hbm,
              cpk, gix, six, sl_src, sl_dst, rows, hist2d, rowidx,
              s_sem, d_sem, gsem, ssem, agg_sh, deg_sh):
    c = lax.axis_index("c")
    s = lax.axis_index("s")
    base = c * HALF
    zv = jnp.zeros((16,), jnp.float32)
    ov = jnp.ones((16,), jnp.float32)
    iot = lax.iota(jnp.int32, 16)

    def zrow(i, _):
        for q in range(2):
            for j in range(16):
                rows[q, i, pl.ds(j * 16, 16)] = zv
        return 0
    lax.fori_loop(0, CHK, zrow, 0)
    for k in range(NPT // ECH):
        pltpu.sync_copy(rows.at[0, pl.ds(0, 48)],
                        agg_sh.at[pl.ds(s * NPT + k * ECH, 48)])
        pltpu.sync_copy(rows.at[1, pl.ds(0, 16)],
                        agg_sh.at[pl.ds(s * NPT + k * ECH + 48, 16)])

    def zh(i, _):
        hist2d[i, pl.ds(0, 16)] = zv
        return 0
    lax.fori_loop(0, 384, zh, 0)
    for r in range(3):
        for j in range(8):
            v = r * 128 + j * 16 + iot
            rowidx[r, pl.ds(j * 16, 16)] = jnp.where(v < NPT, v, NPT)
    pltpu.sync_copy(hist2d.at[pl.ds(0, 20)], deg_sh.at[pl.ds(s * 20, 20)])

    @pl.when(s == 0)
    def _():
        pltpu.sync_copy(rows.at[0, pl.ds(0, 16)], agg_sh.at[pl.ds(DUMP, 16)])
        pltpu.sync_copy(hist2d.at[pl.ds(0, 8)], deg_sh.at[pl.ds(NPT, 8)])

    pltpu.async_copy(src_hbm.at[pl.ds(s * ERT, SLR)], sl_src.at[0],
                     s_sem.at[0])
    pltpu.async_copy(dst_hbm.at[pl.ds(s * ERT, SLR)], sl_dst.at[0],
                     d_sem.at[0])

    dump_row = DUMP + s

    def slab_body(t, cur):
        p = t % 2
        pltpu.make_async_copy(src_hbm.at[pl.ds(s * ERT + t * SLR, SLR)],
                              sl_src.at[p], s_sem.at[p]).wait()
        pltpu.make_async_copy(dst_hbm.at[pl.ds(s * ERT + t * SLR, SLR)],
                              sl_dst.at[p], d_sem.at[p]).wait()

        @pl.when(t + 1 < NSLA)
        def _():
            nt = t + 1
            pltpu.async_copy(
                src_hbm.at[pl.ds(s * ERT + nt * SLR, SLR)],
                sl_src.at[1 - p], s_sem.at[1 - p])
            pltpu.async_copy(
                dst_hbm.at[pl.ds(s * ERT + nt * SLR, SLR)],
                sl_dst.at[1 - p], d_sem.at[1 - p])

        def row_body(r, cur):
            for j in range(ECH // 16):
                sv = sl_src[p, r, pl.ds(j * 16, 16)]
                dv = sl_dst[p, r, pl.ds(j * 16, 16)]
                ld = dv - base
                ok = (ld >= 0) & (ld < HALF)
                ldm = jnp.where(ok, ld, 0)
                plsc.addupdate_scatter(
                    hist2d,
                    [lax.shift_right_logical(ldm, 4), ldm & 15],
                    ov, mask=ok)
                plsc.store_compressed(cpk.at[pl.ds(cur, 16)],
                                      sv | (ld << PKB), mask=ok)
                cur = cur + jnp.sum(ok.astype(jnp.int32))
            return cur
        return lax.fori_loop(0, SLR, row_body, cur)

    cur = lax.fori_loop(0, NSLA, slab_body, jnp.int32(0))

    for j in range(CHK // 16):
        cpk[pl.ds(cur + j * 16, 16)] = jnp.full(
            (16,), 0, jnp.int32) | (dump_row << PKB)
    nch = lax.div(cur + (CHK - 1), CHK)

    plsc.subcore_barrier()

    def unpack(i, q):
        for j in range(CHK // 16):
            v = cpk[pl.ds(i * CHK + j * 16, 16)]
            gix[q, pl.ds(j * 16, 16)] = v & ((1 << PKB) - 1)
            six[q, pl.ds(j * 16, 16)] = lax.shift_right_logical(v, PKB)

    nch = jnp.maximum(nch, 1)
    unpack(0, 0)
    pltpu.async_copy(h_hbm.at[gix.at[0]], rows.at[0], gsem.at[0])

    def gs_body(i, _):
        p = i % 2

        @pl.when(i + 1 < nch)
        def _():
            unpack(i + 1, 1 - p)
            pltpu.async_copy(h_hbm.at[gix.at[1 - p]], rows.at[1 - p],
                             gsem.at[1 - p])
        pltpu.make_async_copy(h_hbm.at[gix.at[p]], rows.at[p],
                              gsem.at[p]).wait()
        pltpu.sync_copy(rows.at[p], agg_sh.at[six.at[p]], add=True)
        return 0
    lax.fori_loop(0, nch, gs_body, 0)

    for r in range(3):
        pltpu.sync_copy(hist2d.at[pl.ds(r * 128, 128)],
                        deg_sh.at[rowidx.at[r]], add=True)

    plsc.subcore_barrier()

    for k in range(NPT // ECH):
        pltpu.sync_copy(agg_sh.at[pl.ds(s * NPT + k * ECH, ECH)],
                        agg_hbm.at[pl.ds(base + s * NPT + k * ECH, ECH)])
    pltpu.sync_copy(deg_sh.at[pl.ds(s * 20, 20)],
                    deg_hbm.at[pl.ds(c * NPT + s * 20, 20)])


def _sc_a(h, src2, dst2):
    return pl.kernel(
        _sca_body,
        out_type=[
            jax.ShapeDtypeStruct((NPAD, D), jnp.float32),
            jax.ShapeDtypeStruct((NPAD // 16, 16), jnp.float32),
        ],
        mesh=_MESH,
        compiler_params=pltpu.CompilerParams(use_tc_tiling_on_sc=False,
                                             needs_layout_passes=False),
        scratch_types=[
            pltpu.VMEM((CMAX,), jnp.int32),
            pltpu.VMEM((2, CHK), jnp.int32),
            pltpu.VMEM((2, CHK), jnp.int32),
            pltpu.VMEM((2, SLR, ECH), jnp.int32),
            pltpu.VMEM((2, SLR, ECH), jnp.int32),
            pltpu.VMEM((2, CHK, D), jnp.float32),
            pltpu.VMEM((384, 16), jnp.float32),
            pltpu.VMEM((3, 128), jnp.int32),
            pltpu.SemaphoreType.DMA((2,)),
            pltpu.SemaphoreType.DMA((2,)),
            pltpu.SemaphoreType.DMA((2,)),
            pltpu.SemaphoreType.DMA((2,)),
            pltpu.VMEM_SHARED((SH_ROWS, D), jnp.float32),
            pltpu.VMEM_SHARED((NPT + 8, 16), jnp.float32),
        ],
    )(h, src2, dst2)


def _scb_body(hp_hbm, ids_hbm, src_hbm, dst_hbm, csum_hbm, cnts_hbm,
              flat_hbm, ids_vm, cflat, rows2, cid_st, fa_src, fa_dst,
              fl_out, fe_st, fa_sem, fb_sem, fl_sem, csum_sh):
    c = lax.axis_index("c")
    s = lax.axis_index("s")
    gt = c * NTS + s
    nb = gt * NPT
    zv = jnp.zeros((16,), jnp.float32)
    ov = jnp.ones((16,), jnp.float32)
    iot = lax.iota(jnp.int32, 16)

    pltpu.sync_copy(ids_hbm, ids_vm)

    def zrow(i, _):
        for j in range(16):
            rows2[i, pl.ds(j * 16, 16)] = zv
        return 0
    lax.fori_loop(0, 64, zrow, 0)
    pltpu.sync_copy(rows2.at[pl.ds(0, 24)], csum_sh.at[pl.ds(s * 24, 24)])

    for i in range(20):
        iv = ids_vm[pl.ds(nb + i * 16, 16)]
        ok = (nb + i * 16 + iot) < N
        cid_st[i // 4, pl.ds((i % 4) * 16, 16)] = jnp.where(ok, iv, C)

    plsc.subcore_barrier()

    for j in range(5):
        pltpu.sync_copy(hp_hbm.at[pl.ds(nb + j * 64, 64)], rows2)
        pltpu.sync_copy(rows2, csum_sh.at[cid_st.at[j]], add=True)

    pltpu.async_copy(src_hbm.at[pl.ds(s * ERT, SLR)], fa_src.at[0],
                     fa_sem.at[0])
    pltpu.async_copy(dst_hbm.at[pl.ds(s * ERT, SLR)], fa_dst.at[0],
                     fb_sem.at[0])

    def fla_body(t, _):
        p = t % 2
        pltpu.make_async_copy(src_hbm.at[pl.ds(s * ERT + t * SLR, SLR)],
                              fa_src.at[p], fa_sem.at[p]).wait()
        pltpu.make_async_copy(dst_hbm.at[pl.ds(s * ERT + t * SLR, SLR)],
                              fa_dst.at[p], fb_sem.at[p]).wait()

        @pl.when(t + 1 < NSLA)
        def _():
            nt = t + 1
            pltpu.async_copy(
                src_hbm.at[pl.ds(s * ERT + nt * SLR, SLR)],
                fa_src.at[1 - p], fa_sem.at[1 - p])
            pltpu.async_copy(
                dst_hbm.at[pl.ds(s * ERT + nt * SLR, SLR)],
                fa_dst.at[1 - p], fb_sem.at[1 - p])

        def fr_body(r, _):
            for j in range(ECH // 16):
                sv = fa_src[p, r, pl.ds(j * 16, 16)]
                dv = fa_dst[p, r, pl.ds(j * 16, 16)]
                cid = plsc.load_gather(ids_vm, [sv])
                fl_out[r, pl.ds(j * 16, 16)] = dv * C + cid
            return 0
        lax.fori_loop(0, SLR, fr_body, 0)
        pltpu.sync_copy(fl_out, flat_hbm.at[pl.ds(s * ERT + t * SLR, SLR)])
        return 0
    lax.fori_loop(0, NSLA, fla_body, 0)

    def zc(i, _):
        cflat[pl.ds(i * 16, 16)] = zv
        return 0
    lax.fori_loop(0, NPT * C // 16, zc, 0)
    for i in range(20):
        iv = ids_vm[pl.ds(nb + i * 16, 16)]
        ok = (nb + i * 16 + iot) < N
        flat = (i * 16 + iot) * C + iv
        plsc.addupdate_scatter(cflat, [flat], ov, mask=ok)

    plsc.subcore_barrier()

    nbc = nb * C
    pltpu.async_copy(flat_hbm.at[pl.ds(0, ESL)], fe_st.at[0], fl_sem.at[0])

    def slab_body(t, _):
        p = t % 2
        pltpu.make_async_copy(flat_hbm.at[pl.ds(t * ESL, ESL)],
                              fe_st.at[p], fl_sem.at[p]).wait()

        @pl.when(t + 1 < NSL)
        def _():
            nt = t + 1
            pltpu.async_copy(flat_hbm.at[pl.ds(nt * ESL, ESL)],
                             fe_st.at[1 - p], fl_sem.at[1 - p])

        def row_body(r, _):
            for j in range(ECH // 16):
                fv = fe_st[p, r, pl.ds(j * 16, 16)]
                ld = fv - nbc
                ok = (ld >= 0) & (ld < NPT * C)
                plsc.addupdate_scatter(cflat, [jnp.where(ok, ld, 0)], ov,
                                       mask=ok)
            return 0
        lax.fori_loop(0, ESL, row_body, 0)
        return 0
    lax.fori_loop(0, NSL, slab_body, 0)

    plsc.subcore_barrier()
    pltpu.sync_copy(csum_sh.at[pl.ds(s * 24, 24)],
                    csum_hbm.at[c, pl.ds(s * 24, 24)])
    pltpu.sync_copy(cflat, cnts_hbm.at[pl.ds(gt * (NPT * C), NPT * C)])


def _sc_b(hp, ids, src2, dst2):
    return pl.kernel(
        _scb_body,
        out_type=[
            jax.ShapeDtypeStruct((2, CSH, D), jnp.float32),
            jax.ShapeDtypeStruct((NPAD * C,), jnp.float32),
            jax.ShapeDtypeStruct((ER, ECH), jnp.int32),
        ],
        mesh=_MESH,
        compiler_params=pltpu.CompilerParams(use_tc_tiling_on_sc=False,
                                             needs_layout_passes=False),
        scratch_types=[
            pltpu.VMEM((NPAD,), jnp.int32),
            pltpu.VMEM((NPT * C,), jnp.float32),
            pltpu.VMEM((64, D), jnp.float32),
            pltpu.VMEM((5, 64), jnp.int32),
            pltpu.VMEM((2, SLR, ECH), jnp.int32),
            pltpu.VMEM((2, SLR, ECH), jnp.int32),
            pltpu.VMEM((SLR, ECH), jnp.int32),
            pltpu.VMEM((2, ESL, ECH), jnp.int32),
            pltpu.SemaphoreType.DMA((2,)),
            pltpu.SemaphoreType.DMA((2,)),
            pltpu.SemaphoreType.DMA((2,)),
            pltpu.VMEM_SHARED((CSH, D), jnp.float32),
        ],
    )(hp, ids, src2, dst2)


def _tc1_body(x_ref, w_ref, b_ref, o_ref):
    o_ref[...] = jnp.dot(x_ref[...], w_ref[...],
                         preferred_element_type=jnp.float32) + b_ref[...]


def _tc1(xp, W1, b1, block_rows=1024):
    grid = (NPAD // block_rows,)
    return pl.pallas_call(
        _tc1_body,
        grid=grid,
        in_specs=[
            pl.BlockSpec((block_rows, D), lambda i: (i, 0)),
            pl.BlockSpec((D, D), lambda i: (0, 0)),
            pl.BlockSpec((D,), lambda i: (0,)),
        ],
        out_specs=pl.BlockSpec((block_rows, D), lambda i: (i, 0)),
        out_shape=jax.ShapeDtypeStruct((NPAD, D), jnp.float32),
    )(xp, W1, b1)


def _tc2_body(agg_ref, h_ref, deg_ref, wsel_ref, hp_ref, ids_ref, cnt_ref):
    i = pl.program_id(0)
    deg = deg_ref[...]
    hp = (agg_ref[...] + h_ref[...]) / (deg[:, None] + 1.0)
    hp_ref[...] = hp
    bits = (hp > 0).astype(jnp.float32)
    idsf = jnp.dot(bits, wsel_ref[...], preferred_element_type=jnp.float32)
    ids = idsf[:, 0].astype(jnp.int32)
    ids_ref[...] = ids
    rows = hp.shape[0]
    gidx = i * rows + lax.broadcasted_iota(jnp.int32, (rows, 1), 0)
    onehot = ((ids[:, None] == lax.broadcasted_iota(jnp.int32, (rows, C), 1))
              & (gidx < N)).astype(jnp.float32)
    part = jnp.sum(onehot, axis=0)

    @pl.when(i == 0)
    def _():
        cnt_ref[...] = jnp.zeros_like(cnt_ref)
    cnt_ref[...] += part


def _tc2(agg, h, deg, wsel, block_rows=1024):
    grid = (NPAD // block_rows,)
    return pl.pallas_call(
        _tc2_body,
        grid=grid,
        in_specs=[
            pl.BlockSpec((block_rows, D), lambda i: (i, 0)),
            pl.BlockSpec((block_rows, D), lambda i: (i, 0)),
            pl.BlockSpec((block_rows,), lambda i: (i,)),
            pl.BlockSpec((D, DOUT), lambda i: (0, 0)),
        ],
        out_specs=[
            pl.BlockSpec((block_rows, D), lambda i: (i, 0)),
            pl.BlockSpec((block_rows,), lambda i: (i,)),
            pl.BlockSpec((C,), lambda i: (0,)),
        ],
        out_shape=[
            jax.ShapeDtypeStruct((NPAD, D), jnp.float32),
            jax.ShapeDtypeStruct((NPAD,), jnp.int32),
            jax.ShapeDtypeStruct((C,), jnp.float32),
        ],
    )(agg, h, deg, wsel)


def _tc3_body(counts_ref, deg_ref, csum_ref, cnt_ref, w2_ref, b2_ref, o_ref):
    csum = csum_ref[0, :C, :] + csum_ref[1, :C, :]
    cnt = cnt_ref[...]
    cmean = csum / jnp.maximum(cnt, 1.0)[:, None]
    z = jnp.dot(cmean, w2_ref[...], preferred_element_type=jnp.float32) \
        + b2_ref[...]
    agg2 = jnp.dot(counts_ref[...], z, preferred_element_type=jnp.float32)
    o_ref[...] = agg2 / (deg_ref[...][:, None] + 1.0)


def _tc3(counts, deg, csum, cnt, W2, b2, block_rows=1024):
    grid = (NPAD // block_rows,)
    return pl.pallas_call(
        _tc3_body,
        grid=grid,
        in_specs=[
            pl.BlockSpec((block_rows, C), lambda i: (i, 0)),
            pl.BlockSpec((block_rows,), lambda i: (i,)),
            pl.BlockSpec((2, CSH, D), lambda i: (0, 0, 0)),
            pl.BlockSpec((C,), lambda i: (0,)),
            pl.BlockSpec((D, DOUT), lambda i: (0, 0)),
            pl.BlockSpec((DOUT,), lambda i: (0,)),
        ],
        out_specs=pl.BlockSpec((block_rows, DOUT), lambda i: (i, 0)),
        out_shape=jax.ShapeDtypeStruct((NPAD, DOUT), jnp.float32),
    )(counts, deg, csum, cnt, W2, b2)


def kernel(x, edge_index, W1, b1, W2, b2):
    xp = jnp.zeros((NPAD, D), jnp.float32).at[:N].set(x)
    src2 = jnp.concatenate(
        [edge_index[0], jnp.zeros((EPAD - E,), jnp.int32)]).reshape(ER, ECH)
    dst2 = jnp.concatenate(
        [edge_index[1], jnp.full((EPAD - E,), NPAD, jnp.int32)]).reshape(
            ER, ECH)
    wsel = jnp.zeros((D, DOUT), jnp.float32).at[:HB, 0].set(
        (2 ** jnp.arange(HB)).astype(jnp.float32))

    h = _tc1(xp, W1, b1)
    agg, deg2 = _sc_a(h, src2, dst2)
    deg = deg2.reshape(NPAD)
    hp, ids, cnt = _tc2(agg, h, deg, wsel)
    csum, cntsf, _flat = _sc_b(hp, ids, src2, dst2)
    counts = cntsf.reshape(NPAD, C)
    out = _tc3(counts, deg, csum, cnt, W2, b2)
    return out[:N]


# speedup vs baseline: 1.0753x; 1.0753x over previous
import jax
import jax.numpy as jnp
from jax import lax
from jax.experimental import pallas as pl
from jax.experimental.pallas import tpu as pltpu
from jax.experimental.pallas import tpu_sc as plsc

N = 10000
E = 160000
D = 256
DOUT = 128
HB = 8
C = 256

NPAD = 10240
HALF = 5120
NTS = 16
SH_ROWS = 5136
DUMP = 5120
CSH = 384
ECH = 64
ER = 2560
EPAD = ER * ECH
ERT = 160
NPT = 320
SLR = 16
NSLA = 10
CHK = 48
CMAX = 10288
PKB = 14
ESL = 32
NSL = 80

_MESH = plsc.VectorSubcoreMesh(core_axis_name="c", subcore_axis_name="s")


def _sca_body(h_hbm, src_hbm, dst_hbm, agg_hbm, deg_hbm,
              cpk, gix, six, sl_src, sl_dst, rows, hist2d, rowidx,
              s_sem, d_sem, gsem, ssem, agg_sh, deg_sh):
    c = lax.axis_index("c")
    s = lax.axis_index("s")
    base = c * HALF
    zv = jnp.zeros((16,), jnp.float32)
    ov = jnp.ones((16,), jnp.float32)
    iot = lax.iota(jnp.int32, 16)

    def zrow(i, _):
        for q in range(2):
            for j in range(16):
                rows[q, i, pl.ds(j * 16, 16)] = zv
        return 0
    lax.fori_loop(0, CHK, zrow, 0)
    for k in range(NPT // ECH):
        pltpu.sync_copy(rows.at[0, pl.ds(0, 48)],
                        agg_sh.at[pl.ds(s * NPT + k * ECH, 48)])
        pltpu.sync_copy(rows.at[1, pl.ds(0, 16)],
                        agg_sh.at[pl.ds(s * NPT + k * ECH + 48, 16)])

    def zh(i, _):
        hist2d[i, pl.ds(0, 16)] = zv
        return 0
    lax.fori_loop(0, 384, zh, 0)
    for r in range(3):
        for j in range(8):
            v = r * 128 + j * 16 + iot
            rowidx[r, pl.ds(j * 16, 16)] = jnp.where(v < NPT, v, NPT)
    pltpu.sync_copy(hist2d.at[pl.ds(0, 20)], deg_sh.at[pl.ds(s * 20, 20)])

    @pl.when(s == 0)
    def _():
        pltpu.sync_copy(rows.at[0, pl.ds(0, 16)], agg_sh.at[pl.ds(DUMP, 16)])
        pltpu.sync_copy(hist2d.at[pl.ds(0, 8)], deg_sh.at[pl.ds(NPT, 8)])

    pltpu.async_copy(src_hbm.at[pl.ds(s * ERT, SLR)], sl_src.at[0],
                     s_sem.at[0])
    pltpu.async_copy(dst_hbm.at[pl.ds(s * ERT, SLR)], sl_dst.at[0],
                     d_sem.at[0])

    dump_row = DUMP + s

    def slab_body(t, cur):
        p = t % 2
        pltpu.make_async_copy(src_hbm.at[pl.ds(s * ERT + t * SLR, SLR)],
                              sl_src.at[p], s_sem.at[p]).wait()
        pltpu.make_async_copy(dst_hbm.at[pl.ds(s * ERT + t * SLR, SLR)],
                              sl_dst.at[p], d_sem.at[p]).wait()

        @pl.when(t + 1 < NSLA)
        def _():
            nt = t + 1
            pltpu.async_copy(
                src_hbm.at[pl.ds(s * ERT + nt * SLR, SLR)],
                sl_src.at[1 - p], s_sem.at[1 - p])
            pltpu.async_copy(
                dst_hbm.at[pl.ds(s * ERT + nt * SLR, SLR)],
                sl_dst.at[1 - p], d_sem.at[1 - p])

        def row_body(r, cur):
            for j in range(ECH // 16):
                sv = sl_src[p, r, pl.ds(j * 16, 16)]
                dv = sl_dst[p, r, pl.ds(j * 16, 16)]
                ld = dv - base
                ok = (ld >= 0) & (ld < HALF)
                ldm = jnp.where(ok, ld, 0)
                plsc.addupdate_scatter(
                    hist2d,
                    [lax.shift_right_logical(ldm, 4), ldm & 15],
                    ov, mask=ok)
                plsc.store_compressed(cpk.at[pl.ds(cur, 16)],
                                      sv | (ld << PKB), mask=ok)
                cur = cur + jnp.sum(ok.astype(jnp.int32))
            return cur
        return lax.fori_loop(0, SLR, row_body, cur)

    cur = lax.fori_loop(0, NSLA, slab_body, jnp.int32(0))

    for j in range(CHK // 16):
        cpk[pl.ds(cur + j * 16, 16)] = jnp.full(
            (16,), 0, jnp.int32) | (dump_row << PKB)
    nch = lax.div(cur + (CHK - 1), CHK)

    plsc.subcore_barrier()

    def unpack(i, q):
        for j in range(CHK // 16):
            v = cpk[pl.ds(i * CHK + j * 16, 16)]
            gix[q, pl.ds(j * 16, 16)] = v & ((1 << PKB) - 1)
            six[q, pl.ds(j * 16, 16)] = lax.shift_right_logical(v, PKB)

    nch = jnp.maximum(nch, 1)
    unpack(0, 0)
    pltpu.async_copy(h_hbm.at[gix.at[0]], rows.at[0], gsem.at[0])

    def gs_body(i, _):
        p = i % 2

        @pl.when(i + 1 < nch)
        def _():
            unpack(i + 1, 1 - p)
            pltpu.async_copy(h_hbm.at[gix.at[1 - p]], rows.at[1 - p],
                             gsem.at[1 - p])
        pltpu.make_async_copy(h_hbm.at[gix.at[p]], rows.at[p],
                              gsem.at[p]).wait()
        pltpu.sync_copy(rows.at[p], agg_sh.at[six.at[p]], add=True)
        return 0
    lax.fori_loop(0, nch, gs_body, 0)

    for r in range(3):
        pltpu.sync_copy(hist2d.at[pl.ds(r * 128, 128)],
                        deg_sh.at[rowidx.at[r]], add=True)

    plsc.subcore_barrier()

    for k in range(NPT // ECH):
        pltpu.sync_copy(agg_sh.at[pl.ds(s * NPT + k * ECH, ECH)],
                        agg_hbm.at[pl.ds(base + s * NPT + k * ECH, ECH)])
    pltpu.sync_copy(deg_sh.at[pl.ds(s * 20, 20)],
                    deg_hbm.at[pl.ds(c * NPT + s * 20, 20)])


def _sc_a(h, src2, dst2):
    return pl.kernel(
        _sca_body,
        out_type=[
            jax.ShapeDtypeStruct((NPAD, D), jnp.float32),
            jax.ShapeDtypeStruct((NPAD // 16, 16), jnp.float32),
        ],
        mesh=_MESH,
        compiler_params=pltpu.CompilerParams(use_tc_tiling_on_sc=False,
                                             needs_layout_passes=False),
        scratch_types=[
            pltpu.VMEM((CMAX,), jnp.int32),
            pltpu.VMEM((2, CHK), jnp.int32),
            pltpu.VMEM((2, CHK), jnp.int32),
            pltpu.VMEM((2, SLR, ECH), jnp.int32),
            pltpu.VMEM((2, SLR, ECH), jnp.int32),
            pltpu.VMEM((2, CHK, D), jnp.float32),
            pltpu.VMEM((384, 16), jnp.float32),
            pltpu.VMEM((3, 128), jnp.int32),
            pltpu.SemaphoreType.DMA((2,)),
            pltpu.SemaphoreType.DMA((2,)),
            pltpu.SemaphoreType.DMA((2,)),
            pltpu.SemaphoreType.DMA((2,)),
            pltpu.VMEM_SHARED((SH_ROWS, D), jnp.float32),
            pltpu.VMEM_SHARED((NPT + 8, 16), jnp.float32),
        ],
    )(h, src2, dst2)


def _scb_body(hp_hbm, ids_hbm, src_hbm, dst_hbm, csum_hbm, cnts_hbm,
              flat_hbm, ids_vm, cflat, rows2, cid_st, fa_src, fa_dst,
              fl_out, fe_st, fa_sem, fb_sem, fl_sem, csum_sh):
    c = lax.axis_index("c")
    s = lax.axis_index("s")
    gt = c * NTS + s
    nb = gt * NPT
    zv = jnp.zeros((16,), jnp.float32)
    ov = jnp.ones((16,), jnp.float32)
    iot = lax.iota(jnp.int32, 16)

    pltpu.sync_copy(ids_hbm, ids_vm)

    def zrow(i, _):
        for j in range(16):
            rows2[i, pl.ds(j * 16, 16)] = zv
        return 0
    lax.fori_loop(0, 64, zrow, 0)
    pltpu.sync_copy(rows2.at[pl.ds(0, 24)], csum_sh.at[pl.ds(s * 24, 24)])

    for i in range(20):
        iv = ids_vm[pl.ds(nb + i * 16, 16)]
        ok = (nb + i * 16 + iot) < N
        cid_st[i // 4, pl.ds((i % 4) * 16, 16)] = jnp.where(ok, iv, C)

    plsc.subcore_barrier()

    for j in range(5):
        pltpu.sync_copy(hp_hbm.at[pl.ds(nb + j * 64, 64)], rows2)
        pltpu.sync_copy(rows2, csum_sh.at[cid_st.at[j]], add=True)

    pltpu.async_copy(src_hbm.at[pl.ds(s * ERT, SLR)], fa_src.at[0],
                     fa_sem.at[0])
    pltpu.async_copy(dst_hbm.at[pl.ds(s * ERT, SLR)], fa_dst.at[0],
                     fb_sem.at[0])

    def fla_body(t, _):
        p = t % 2
        pltpu.make_async_copy(src_hbm.at[pl.ds(s * ERT + t * SLR, SLR)],
                              fa_src.at[p], fa_sem.at[p]).wait()
        pltpu.make_async_copy(dst_hbm.at[pl.ds(s * ERT + t * SLR, SLR)],
                              fa_dst.at[p], fb_sem.at[p]).wait()

        @pl.when(t + 1 < NSLA)
        def _():
            nt = t + 1
            pltpu.async_copy(
                src_hbm.at[pl.ds(s * ERT + nt * SLR, SLR)],
                fa_src.at[1 - p], fa_sem.at[1 - p])
            pltpu.async_copy(
                dst_hbm.at[pl.ds(s * ERT + nt * SLR, SLR)],
                fa_dst.at[1 - p], fb_sem.at[1 - p])

        def fr_body(r, _):
            for j in range(ECH // 16):
                sv = fa_src[p, r, pl.ds(j * 16, 16)]
                dv = fa_dst[p, r, pl.ds(j * 16, 16)]
                cid = plsc.load_gather(ids_vm, [sv])
                fl_out[r, pl.ds(j * 16, 16)] = dv * C + cid
            return 0
        lax.fori_loop(0, SLR, fr_body, 0)
        pltpu.sync_copy(fl_out, flat_hbm.at[pl.ds(s * ERT + t * SLR, SLR)])
        return 0
    lax.fori_loop(0, NSLA, fla_body, 0)

    def zc(i, _):
        for j in range(8):
            cflat[pl.ds(i * 128 + j * 16, 16)] = zv
        return 0
    lax.fori_loop(0, NPT * C // 128, zc, 0)
    for i in range(20):
        iv = ids_vm[pl.ds(nb + i * 16, 16)]
        ok = (nb + i * 16 + iot) < N
        flat = (i * 16 + iot) * C + iv
        plsc.addupdate_scatter(cflat, [flat], ov, mask=ok)

    plsc.subcore_barrier()

    nbc = nb * C
    pltpu.async_copy(flat_hbm.at[pl.ds(0, ESL)], fe_st.at[0], fl_sem.at[0])

    def slab_body(t, _):
        p = t % 2
        pltpu.make_async_copy(flat_hbm.at[pl.ds(t * ESL, ESL)],
                              fe_st.at[p], fl_sem.at[p]).wait()

        @pl.when(t + 1 < NSL)
        def _():
            nt = t + 1
            pltpu.async_copy(flat_hbm.at[pl.ds(nt * ESL, ESL)],
                             fe_st.at[1 - p], fl_sem.at[1 - p])

        def row_body(r, _):
            for j in range(ECH // 16):
                fv = fe_st[p, r, pl.ds(j * 16, 16)]
                ld = fv - nbc
                ok = (ld >= 0) & (ld < NPT * C)
                plsc.addupdate_scatter(cflat, [jnp.where(ok, ld, 0)], ov,
                                       mask=ok)
            return 0
        lax.fori_loop(0, ESL, row_body, 0)
        return 0
    lax.fori_loop(0, NSL, slab_body, 0)

    plsc.subcore_barrier()
    pltpu.sync_copy(csum_sh.at[pl.ds(s * 24, 24)],
                    csum_hbm.at[c, pl.ds(s * 24, 24)])
    pltpu.sync_copy(cflat, cnts_hbm.at[pl.ds(gt * (NPT * C), NPT * C)])


def _sc_b(hp, ids, src2, dst2):
    return pl.kernel(
        _scb_body,
        out_type=[
            jax.ShapeDtypeStruct((2, CSH, D), jnp.float32),
            jax.ShapeDtypeStruct((NPAD * C,), jnp.float32),
            jax.ShapeDtypeStruct((ER, ECH), jnp.int32),
        ],
        mesh=_MESH,
        compiler_params=pltpu.CompilerParams(use_tc_tiling_on_sc=False,
                                             needs_layout_passes=False),
        scratch_types=[
            pltpu.VMEM((NPAD,), jnp.int32),
            pltpu.VMEM((NPT * C,), jnp.float32),
            pltpu.VMEM((64, D), jnp.float32),
            pltpu.VMEM((5, 64), jnp.int32),
            pltpu.VMEM((2, SLR, ECH), jnp.int32),
            pltpu.VMEM((2, SLR, ECH), jnp.int32),
            pltpu.VMEM((SLR, ECH), jnp.int32),
            pltpu.VMEM((2, ESL, ECH), jnp.int32),
            pltpu.SemaphoreType.DMA((2,)),
            pltpu.SemaphoreType.DMA((2,)),
            pltpu.SemaphoreType.DMA((2,)),
            pltpu.VMEM_SHARED((CSH, D), jnp.float32),
        ],
    )(hp, ids, src2, dst2)


def _tc1_body(x_ref, w_ref, b_ref, o_ref):
    o_ref[...] = jnp.dot(x_ref[...], w_ref[...],
                         preferred_element_type=jnp.float32) + b_ref[...]


def _tc1(x, W1, b1, block_rows=1024):
    grid = (NPAD // block_rows,)
    return pl.pallas_call(
        _tc1_body,
        grid=grid,
        in_specs=[
            pl.BlockSpec((block_rows, D), lambda i: (i, 0)),
            pl.BlockSpec((D, D), lambda i: (0, 0)),
            pl.BlockSpec((D,), lambda i: (0,)),
        ],
        out_specs=pl.BlockSpec((block_rows, D), lambda i: (i, 0)),
        out_shape=jax.ShapeDtypeStruct((NPAD, D), jnp.float32),
    )(x, W1, b1)


def _tc2_body(agg_ref, h_ref, deg_ref, wsel_ref, hp_ref, ids_ref, cnt_ref):
    i = pl.program_id(0)
    deg = deg_ref[...]
    hp = (agg_ref[...] + h_ref[...]) / (deg[:, None] + 1.0)
    hp_ref[...] = hp
    bits = (hp > 0).astype(jnp.float32)
    idsf = jnp.dot(bits, wsel_ref[...], preferred_element_type=jnp.float32)
    ids = idsf[:, 0].astype(jnp.int32)
    ids_ref[...] = ids
    rows = hp.shape[0]
    gidx = i * rows + lax.broadcasted_iota(jnp.int32, (rows, 1), 0)
    onehot = ((ids[:, None] == lax.broadcasted_iota(jnp.int32, (rows, C), 1))
              & (gidx < N)).astype(jnp.float32)
    part = jnp.sum(onehot, axis=0)

    @pl.when(i == 0)
    def _():
        cnt_ref[...] = jnp.zeros_like(cnt_ref)
    cnt_ref[...] += part


def _tc2(agg, h, deg, wsel, block_rows=1024):
    grid = (NPAD // block_rows,)
    return pl.pallas_call(
        _tc2_body,
        grid=grid,
        in_specs=[
            pl.BlockSpec((block_rows, D), lambda i: (i, 0)),
            pl.BlockSpec((block_rows, D), lambda i: (i, 0)),
            pl.BlockSpec((block_rows,), lambda i: (i,)),
            pl.BlockSpec((D, DOUT), lambda i: (0, 0)),
        ],
        out_specs=[
            pl.BlockSpec((block_rows, D), lambda i: (i, 0)),
            pl.BlockSpec((block_rows,), lambda i: (i,)),
            pl.BlockSpec((C,), lambda i: (0,)),
        ],
        out_shape=[
            jax.ShapeDtypeStruct((NPAD, D), jnp.float32),
            jax.ShapeDtypeStruct((NPAD,), jnp.int32),
            jax.ShapeDtypeStruct((C,), jnp.float32),
        ],
    )(agg, h, deg, wsel)


def _tc3_body(counts_ref, deg_ref, csum_ref, cnt_ref, w2_ref, b2_ref, o_ref):
    csum = csum_ref[0, :C, :] + csum_ref[1, :C, :]
    cnt = cnt_ref[...]
    cmean = csum / jnp.maximum(cnt, 1.0)[:, None]
    z = jnp.dot(cmean, w2_ref[...], preferred_element_type=jnp.float32) \
        + b2_ref[...]
    agg2 = jnp.dot(counts_ref[...], z, preferred_element_type=jnp.float32)
    o_ref[...] = agg2 / (deg_ref[...][:, None] + 1.0)


def _tc3(counts, deg, csum, cnt, W2, b2, block_rows=1024):
    grid = (NPAD // block_rows,)
    return pl.pallas_call(
        _tc3_body,
        grid=grid,
        in_specs=[
            pl.BlockSpec((block_rows, C), lambda i: (i, 0)),
            pl.BlockSpec((block_rows,), lambda i: (i,)),
            pl.BlockSpec((2, CSH, D), lambda i: (0, 0, 0)),
            pl.BlockSpec((C,), lambda i: (0,)),
            pl.BlockSpec((D, DOUT), lambda i: (0, 0)),
            pl.BlockSpec((DOUT,), lambda i: (0,)),
        ],
        out_specs=pl.BlockSpec((block_rows, DOUT), lambda i: (i, 0)),
        out_shape=jax.ShapeDtypeStruct((NPAD, DOUT), jnp.float32),
    )(counts, deg, csum, cnt, W2, b2)


def kernel(x, edge_index, W1, b1, W2, b2):
    src2 = jnp.concatenate(
        [edge_index[0], jnp.zeros((EPAD - E,), jnp.int32)]).reshape(ER, ECH)
    dst2 = jnp.concatenate(
        [edge_index[1], jnp.full((EPAD - E,), NPAD, jnp.int32)]).reshape(
            ER, ECH)
    wsel = jnp.zeros((D, DOUT), jnp.float32).at[:HB, 0].set(
        (2 ** jnp.arange(HB)).astype(jnp.float32))

    h = _tc1(x, W1, b1)
    agg, deg2 = _sc_a(h, src2, dst2)
    deg = deg2.reshape(NPAD)
    hp, ids, cnt = _tc2(agg, h, deg, wsel)
    csum, cntsf, _flat = _sc_b(hp, ids, src2, dst2)
    counts = cntsf.reshape(NPAD, C)
    out = _tc3(counts, deg, csum, cnt, W2, b2)
    return out[:N]
